# baseline (speedup 1.0000x reference)
import functools

import jax
import jax.numpy as jnp
from jax import lax
from jax.experimental import pallas as pl
from jax.experimental.pallas import tpu as pltpu
from jax.experimental.pallas import tpu_sc as plsc

N_NODES = 10000
NPAD = 10240
IN_CH = 128
HID = 256
OUT_CH = 128
N_EDGES = 320000

NC = 2
NS = 16
CHUNK = 128
DEG_CHUNKS = 80
AGG_CHUNKS = 160
IDX_BLK = 40
N_IDXBLK = AGG_CHUNKS // IDX_BLK
EPAD = NC * NS * DEG_CHUNKS * CHUNK
ROWS_PER_TILE = NPAD // NS

_MESH = plsc.VectorSubcoreMesh(core_axis_name="c", subcore_axis_name="s")


EPW = EPAD // (NC * NS)
_LANES = 16


@functools.partial(
    pl.kernel,
    out_type=jax.ShapeDtypeStruct((NC * NS, NPAD), jnp.float32),
    mesh=_MESH,
    scratch_types=[
        pltpu.VMEM((EPW,), jnp.int32),
        pltpu.VMEM((NPAD,), jnp.float32),
    ],
    compiler_params=pltpu.CompilerParams(needs_layout_passes=False),
)
def _deg_kernel(dst_hbm, zeros_hbm, out_hbm, idx_v, hist_v):
    c = lax.axis_index("c")
    s = lax.axis_index("s")
    w = s * NC + c
    pltpu.sync_copy(dst_hbm.at[w], idx_v)
    pltpu.sync_copy(zeros_hbm, hist_v)
    ones = jnp.ones((_LANES,), jnp.float32)

    def body(i, carry):
        idx = idx_v[pl.ds(i * _LANES, _LANES)]
        plsc.addupdate_scatter(hist_v, [idx], ones)
        return carry

    lax.fori_loop(0, EPW // _LANES, body, 0)
    pltpu.sync_copy(hist_v, out_hbm.at[w])


def _make_agg(nblk, core_stride, offset):
    @functools.partial(
        pl.kernel,
        out_type=jax.ShapeDtypeStruct((NC, NPAD, 128), jnp.float32),
        mesh=_MESH,
        scratch_types=[
            pltpu.VMEM((IDX_BLK, CHUNK), jnp.int32),
            pltpu.VMEM((IDX_BLK, CHUNK), jnp.int32),
            pltpu.VMEM((CHUNK, 128), jnp.float32),
            pltpu.VMEM((CHUNK, 128), jnp.float32),
            pltpu.VMEM_SHARED((NPAD, 128), jnp.float32),
            pltpu.SemaphoreType.DMA,
            pltpu.SemaphoreType.DMA,
            pltpu.SemaphoreType.DMA,
            pltpu.SemaphoreType.DMA,
        ],
    )
    def agg(src_hbm, dst_hbm, y_hbm, out_hbm,
            src_v, dst_v, rows0, rows1, acc_sh, sem0, sem1, ssem0, ssem1):
        c = lax.axis_index("c")
        s = lax.axis_index("s")
        y_c = y_hbm.at[c]
        idx_src = src_hbm.at[s]
        idx_dst = dst_hbm.at[s]
        r0 = s * ROWS_PER_TILE
        pltpu.sync_copy(y_c.at[pl.ds(r0, ROWS_PER_TILE)],
                        acc_sh.at[pl.ds(r0, ROWS_PER_TILE)])
        plsc.subcore_barrier()

        def outer(bi, carry):
            blk = offset + c * core_stride + bi
            pltpu.sync_copy(idx_src.at[pl.ds(blk * IDX_BLK, IDX_BLK)], src_v)
            pltpu.sync_copy(idx_dst.at[pl.ds(blk * IDX_BLK, IDX_BLK)], dst_v)

            pltpu.async_copy(y_c.at[src_v.at[0]], rows0, sem0)
            pltpu.async_copy(y_c.at[src_v.at[1]], rows1, sem1)

            def pair(p, carry2):
                j = 2 * p
                pltpu.make_async_copy(y_c.at[src_v.at[j]], rows0, sem0).wait()
                pltpu.async_copy(rows0, acc_sh.at[dst_v.at[j]], ssem0, add=True)
                pltpu.make_async_copy(y_c.at[src_v.at[j + 1]], rows1, sem1).wait()
                pltpu.async_copy(rows1, acc_sh.at[dst_v.at[j + 1]], ssem1, add=True)
                pltpu.make_async_copy(rows0, acc_sh.at[dst_v.at[j]], ssem0).wait()
                pltpu.async_copy(y_c.at[src_v.at[j + 2]], rows0, sem0)
                pltpu.make_async_copy(rows1, acc_sh.at[dst_v.at[j + 1]], ssem1).wait()
                pltpu.async_copy(y_c.at[src_v.at[j + 3]], rows1, sem1)
                return carry2

            lax.fori_loop(0, IDX_BLK // 2 - 1, pair, 0)
            jl = IDX_BLK - 2
            pltpu.make_async_copy(y_c.at[src_v.at[jl]], rows0, sem0).wait()
            pltpu.sync_copy(rows0, acc_sh.at[dst_v.at[jl]], add=True)
            pltpu.make_async_copy(y_c.at[src_v.at[jl + 1]], rows1, sem1).wait()
            pltpu.sync_copy(rows1, acc_sh.at[dst_v.at[jl + 1]], add=True)
            return carry

        lax.fori_loop(0, nblk, outer, 0)
        plsc.subcore_barrier()
        pltpu.sync_copy(acc_sh.at[pl.ds(r0, ROWS_PER_TILE)],
                        out_hbm.at[c].at[pl.ds(r0, ROWS_PER_TILE)])

    return agg


_agg_l1 = _make_agg(nblk=4, core_stride=0, offset=0)
_agg_l2 = _make_agg(nblk=2, core_stride=2, offset=0)


_RB = 512
_GRID = NPAD // _RB


def _dis_from(dp_ref):
    deg = jnp.sum(dp_ref[...], axis=0)[:, None] + 1.0
    return lax.rsqrt(deg)


def _xw_body(x_ref, w_ref, dp_ref, o_ref):
    dis = _dis_from(dp_ref)
    xw = jnp.dot(x_ref[...], w_ref[...], preferred_element_type=jnp.float32)
    y = xw * dis
    h = w_ref.shape[1] // 2
    o_ref[0] = y[:, :h]
    o_ref[1] = y[:, h:]


def _mid_body(a_ref, dp_ref, w_ref, b_ref, o_ref):
    dis = _dis_from(dp_ref)
    aggf = jnp.concatenate([a_ref[0], a_ref[1]], axis=1)
    h = jnp.maximum(aggf * dis + b_ref[...], 0.0)
    hw = jnp.dot(h, w_ref[...], preferred_element_type=jnp.float32)
    y2 = hw * dis
    o_ref[0] = y2
    o_ref[1] = y2


def _fin_body(p_ref, y2_ref, dp_ref, b_ref, o_ref):
    dis = _dis_from(dp_ref)
    aggf = p_ref[0] + p_ref[1] - y2_ref[...]
    o_ref[...] = aggf * dis + b_ref[...]


def _tc_xw(x, w, degp, d_out):
    return pl.pallas_call(
        _xw_body,
        grid=(_GRID,),
        in_specs=[
            pl.BlockSpec((_RB, x.shape[1]), lambda i: (i, 0)),
            pl.BlockSpec((w.shape[0], w.shape[1]), lambda i: (0, 0)),
            pl.BlockSpec((NC * NS, _RB), lambda i: (0, i)),
        ],
        out_specs=pl.BlockSpec((2, _RB, d_out // 2), lambda i: (0, i, 0)),
        out_shape=jax.ShapeDtypeStruct((2, NPAD, d_out // 2), jnp.float32),
    )(x, w, degp)


def _tc_mid(agg1, degp, w, b, d_out):
    return pl.pallas_call(
        _mid_body,
        grid=(_GRID,),
        in_specs=[
            pl.BlockSpec((2, _RB, 128), lambda i: (0, i, 0)),
            pl.BlockSpec((NC * NS, _RB), lambda i: (0, i)),
            pl.BlockSpec((w.shape[0], w.shape[1]), lambda i: (0, 0)),
            pl.BlockSpec((1, b.shape[1]), lambda i: (0, 0)),
        ],
        out_specs=pl.BlockSpec((2, _RB, d_out), lambda i: (0, i, 0)),
        out_shape=jax.ShapeDtypeStruct((2, NPAD, d_out), jnp.float32),
    )(agg1, degp, w, b)


def _tc_fin(parts, y2, degp, b):
    d = y2.shape[1]
    return pl.pallas_call(
        _fin_body,
        grid=(_GRID,),
        in_specs=[
            pl.BlockSpec((2, _RB, d), lambda i: (0, i, 0)),
            pl.BlockSpec((_RB, d), lambda i: (i, 0)),
            pl.BlockSpec((NC * NS, _RB), lambda i: (0, i)),
            pl.BlockSpec((1, d), lambda i: (0, 0)),
        ],
        out_specs=pl.BlockSpec((_RB, d), lambda i: (i, 0)),
        out_shape=jax.ShapeDtypeStruct((NPAD, d), jnp.float32),
    )(parts, y2, degp, b)


def kernel(x, edge_index, W1, b1, W2, b2):
    src = edge_index[0].astype(jnp.int32)
    dst = edge_index[1].astype(jnp.int32)
    npad_e = EPAD - N_EDGES
    pad_dst = N_NODES + (jnp.arange(npad_e, dtype=jnp.int32) % (NPAD - N_NODES))
    src_p = jnp.concatenate([src, jnp.zeros((npad_e,), jnp.int32)])
    dst_p = jnp.concatenate([dst, pad_dst])
    src_agg = src_p.reshape(NS, AGG_CHUNKS, CHUNK)
    dst_agg = dst_p.reshape(NS, AGG_CHUNKS, CHUNK)
    dst_deg = dst_p.reshape(NC * NS, EPW)

    zeros1 = jnp.zeros((NPAD,), jnp.float32)
    xp = jnp.zeros((NPAD, IN_CH), x.dtype).at[:N_NODES].set(x)

    degp = _deg_kernel(dst_deg, zeros1)

    y1 = _tc_xw(xp, W1, degp, HID)
    agg1 = _agg_l1(src_agg, dst_agg, y1)
    y2 = _tc_mid(agg1, degp, W2, b1.reshape(1, HID), OUT_CH)
    parts = _agg_l2(src_agg, dst_agg, y2)
    z = _tc_fin(parts, y2[0], degp, b2.reshape(1, OUT_CH))
    return z[:N_NODES]


# speedup vs baseline: 1.1999x; 1.1999x over previous
import functools

import jax
import jax.numpy as jnp
from jax import lax
from jax.experimental import pallas as pl
from jax.experimental.pallas import tpu as pltpu
from jax.experimental.pallas import tpu_sc as plsc

N_NODES = 10000
NPAD = 10240
IN_CH = 128
HID = 256
OUT_CH = 128
N_EDGES = 320000

NC = 2
NS = 16
CHUNK = 128
DEG_CHUNKS = 80
AGG_CHUNKS = 160
IDX_BLK = 40
N_IDXBLK = AGG_CHUNKS // IDX_BLK
EPAD = NC * NS * DEG_CHUNKS * CHUNK
ROWS_PER_TILE = NPAD // NS

_MESH = plsc.VectorSubcoreMesh(core_axis_name="c", subcore_axis_name="s")


EPW = EPAD // (NC * NS)
_LANES = 16


@functools.partial(
    pl.kernel,
    out_type=jax.ShapeDtypeStruct((NC * NS, NPAD), jnp.float32),
    mesh=_MESH,
    scratch_types=[
        pltpu.VMEM((EPW,), jnp.int32),
        pltpu.VMEM((NPAD,), jnp.float32),
    ],
    compiler_params=pltpu.CompilerParams(needs_layout_passes=False),
)
def _deg_kernel(dst_hbm, zeros_hbm, out_hbm, idx_v, hist_v):
    c = lax.axis_index("c")
    s = lax.axis_index("s")
    w = s * NC + c
    pltpu.sync_copy(dst_hbm.at[w], idx_v)
    pltpu.sync_copy(zeros_hbm, hist_v)
    ones = jnp.ones((_LANES,), jnp.float32)

    def body(i, carry):
        idx = idx_v[pl.ds(i * _LANES, _LANES)]
        plsc.addupdate_scatter(hist_v, [idx], ones)
        return carry

    lax.fori_loop(0, EPW // _LANES, body, 0)
    pltpu.sync_copy(hist_v, out_hbm.at[w])


def _make_agg(nblk, core_stride, offset, bi_stride=1):
    @functools.partial(
        pl.kernel,
        out_type=jax.ShapeDtypeStruct((NC, NPAD, 128), jnp.float32),
        mesh=_MESH,
        scratch_types=[
            pltpu.VMEM((IDX_BLK, CHUNK), jnp.int32),
            pltpu.VMEM((IDX_BLK, CHUNK), jnp.int32),
            pltpu.VMEM((CHUNK, 128), jnp.float32),
            pltpu.VMEM((CHUNK, 128), jnp.float32),
            pltpu.VMEM_SHARED((NPAD, 128), jnp.float32),
            pltpu.SemaphoreType.DMA,
            pltpu.SemaphoreType.DMA,
        ],
    )
    def agg(src_hbm, dst_hbm, y_hbm, out_hbm,
            src_v, dst_v, rows0, rows1, acc_sh, sem0, sem1):
        c = lax.axis_index("c")
        s = lax.axis_index("s")
        y_c = y_hbm.at[c]
        idx_src = src_hbm.at[s]
        idx_dst = dst_hbm.at[s]
        r0 = s * ROWS_PER_TILE
        pltpu.sync_copy(y_c.at[pl.ds(r0, ROWS_PER_TILE)],
                        acc_sh.at[pl.ds(r0, ROWS_PER_TILE)])
        plsc.subcore_barrier()

        def outer(bi, carry):
            blk = offset + c * core_stride + bi * bi_stride
            pltpu.sync_copy(idx_src.at[pl.ds(blk * IDX_BLK, IDX_BLK)], src_v)
            pltpu.sync_copy(idx_dst.at[pl.ds(blk * IDX_BLK, IDX_BLK)], dst_v)

            pltpu.async_copy(y_c.at[src_v.at[0]], rows0, sem0)
            pltpu.async_copy(y_c.at[src_v.at[1]], rows1, sem1)

            def pair(p, carry2):
                j = 2 * p
                pltpu.make_async_copy(y_c.at[src_v.at[j]], rows0, sem0).wait()
                pltpu.sync_copy(rows0, acc_sh.at[dst_v.at[j]], add=True)
                pltpu.async_copy(y_c.at[src_v.at[j + 2]], rows0, sem0)
                pltpu.make_async_copy(y_c.at[src_v.at[j + 1]], rows1, sem1).wait()
                pltpu.sync_copy(rows1, acc_sh.at[dst_v.at[j + 1]], add=True)
                pltpu.async_copy(y_c.at[src_v.at[j + 3]], rows1, sem1)
                return carry2

            lax.fori_loop(0, IDX_BLK // 2 - 1, pair, 0)
            jl = IDX_BLK - 2
            pltpu.make_async_copy(y_c.at[src_v.at[jl]], rows0, sem0).wait()
            pltpu.sync_copy(rows0, acc_sh.at[dst_v.at[jl]], add=True)
            pltpu.make_async_copy(y_c.at[src_v.at[jl + 1]], rows1, sem1).wait()
            pltpu.sync_copy(rows1, acc_sh.at[dst_v.at[jl + 1]], add=True)
            return carry

        lax.fori_loop(0, nblk, outer, 0)
        plsc.subcore_barrier()
        pltpu.sync_copy(acc_sh.at[pl.ds(r0, ROWS_PER_TILE)],
                        out_hbm.at[c].at[pl.ds(r0, ROWS_PER_TILE)])

    return agg


_agg_l1 = _make_agg(nblk=4, core_stride=0, offset=0)
_agg_l2 = _make_agg(nblk=2, core_stride=1, offset=0, bi_stride=2)


_RB = 512
_GRID = NPAD // _RB


def _dis_from(dp_ref):
    deg = jnp.sum(dp_ref[...], axis=0)[:, None] + 1.0
    return lax.rsqrt(deg)


def _xw_body(x_ref, w_ref, dp_ref, o_ref):
    dis = _dis_from(dp_ref)
    xw = jnp.dot(x_ref[...], w_ref[...], preferred_element_type=jnp.float32)
    y = xw * dis
    h = w_ref.shape[1] // 2
    o_ref[0] = y[:, :h]
    o_ref[1] = y[:, h:]


def _mid_body(a_ref, dp_ref, w_ref, b_ref, o_ref):
    dis = _dis_from(dp_ref)
    aggf = jnp.concatenate([a_ref[0], a_ref[1]], axis=1)
    h = jnp.maximum(aggf * dis + b_ref[...], 0.0)
    hw = jnp.dot(h, w_ref[...], preferred_element_type=jnp.float32)
    y2 = hw * dis
    o_ref[0] = y2
    o_ref[1] = y2


def _fin_body(p_ref, y2_ref, dp_ref, b_ref, o_ref):
    dis = _dis_from(dp_ref)
    aggf = p_ref[0] + p_ref[1] - y2_ref[...]
    o_ref[...] = aggf * dis + b_ref[...]


def _tc_xw(x, w, degp, d_out):
    return pl.pallas_call(
        _xw_body,
        grid=(_GRID,),
        in_specs=[
            pl.BlockSpec((_RB, x.shape[1]), lambda i: (i, 0)),
            pl.BlockSpec((w.shape[0], w.shape[1]), lambda i: (0, 0)),
            pl.BlockSpec((NC * NS, _RB), lambda i: (0, i)),
        ],
        out_specs=pl.BlockSpec((2, _RB, d_out // 2), lambda i: (0, i, 0)),
        out_shape=jax.ShapeDtypeStruct((2, NPAD, d_out // 2), jnp.float32),
    )(x, w, degp)


def _tc_mid(agg1, degp, w, b, d_out):
    return pl.pallas_call(
        _mid_body,
        grid=(_GRID,),
        in_specs=[
            pl.BlockSpec((2, _RB, 128), lambda i: (0, i, 0)),
            pl.BlockSpec((NC * NS, _RB), lambda i: (0, i)),
            pl.BlockSpec((w.shape[0], w.shape[1]), lambda i: (0, 0)),
            pl.BlockSpec((1, b.shape[1]), lambda i: (0, 0)),
        ],
        out_specs=pl.BlockSpec((2, _RB, d_out), lambda i: (0, i, 0)),
        out_shape=jax.ShapeDtypeStruct((2, NPAD, d_out), jnp.float32),
    )(agg1, degp, w, b)


def _tc_fin(parts, y2, degp, b):
    d = y2.shape[1]
    return pl.pallas_call(
        _fin_body,
        grid=(_GRID,),
        in_specs=[
            pl.BlockSpec((2, _RB, d), lambda i: (0, i, 0)),
            pl.BlockSpec((_RB, d), lambda i: (i, 0)),
            pl.BlockSpec((NC * NS, _RB), lambda i: (0, i)),
            pl.BlockSpec((1, d), lambda i: (0, 0)),
        ],
        out_specs=pl.BlockSpec((_RB, d), lambda i: (i, 0)),
        out_shape=jax.ShapeDtypeStruct((NPAD, d), jnp.float32),
    )(parts, y2, degp, b)


def kernel(x, edge_index, W1, b1, W2, b2):
    src = edge_index[0].astype(jnp.int32)
    dst = edge_index[1].astype(jnp.int32)
    npad_e = EPAD - N_EDGES
    pad_dst = N_NODES + (jnp.arange(npad_e, dtype=jnp.int32) % (NPAD - N_NODES))
    src_p = jnp.concatenate([src, jnp.zeros((npad_e,), jnp.int32)])
    dst_p = jnp.concatenate([dst, pad_dst])
    src_agg = src_p.reshape(NS, AGG_CHUNKS, CHUNK)
    dst_agg = dst_p.reshape(NS, AGG_CHUNKS, CHUNK)
    dst_deg = dst_p.reshape(NC * NS, EPW)

    zeros1 = jnp.zeros((NPAD,), jnp.float32)
    xp = jnp.zeros((NPAD, IN_CH), x.dtype).at[:N_NODES].set(x)

    degp = _deg_kernel(dst_deg, zeros1)

    y1 = _tc_xw(xp, W1, degp, HID)
    agg1 = _agg_l1(src_agg, dst_agg, y1)
    y2 = _tc_mid(agg1, degp, W2, b1.reshape(1, HID), OUT_CH)
    parts = _agg_l2(src_agg, dst_agg, y2)
    z = _tc_fin(parts, y2[0], degp, b2.reshape(1, OUT_CH))
    return z[:N_NODES]


# speedup vs baseline: 1.2701x; 1.0585x over previous
import functools

import jax
import jax.numpy as jnp
from jax import lax
from jax.experimental import pallas as pl
from jax.experimental.pallas import tpu as pltpu
from jax.experimental.pallas import tpu_sc as plsc

N_NODES = 10000
NPAD = 10240
IN_CH = 128
HID = 256
OUT_CH = 128
N_EDGES = 320000

NC = 2
NS = 16
CHUNK = 128
DEG_CHUNKS = 80
AGG_CHUNKS = 160
IDX_BLK = 40
N_IDXBLK = AGG_CHUNKS // IDX_BLK
EPAD = NC * NS * DEG_CHUNKS * CHUNK
ROWS_PER_TILE = NPAD // NS

_MESH = plsc.VectorSubcoreMesh(core_axis_name="c", subcore_axis_name="s")


EPW = EPAD // (NC * NS)
_LANES = 16


@functools.partial(
    pl.kernel,
    out_type=jax.ShapeDtypeStruct((NC * NS, NPAD), jnp.float32),
    mesh=_MESH,
    scratch_types=[
        pltpu.VMEM((EPW,), jnp.int32),
        pltpu.VMEM((NPAD,), jnp.float32),
    ],
    compiler_params=pltpu.CompilerParams(needs_layout_passes=False),
)
def _deg_kernel(dst_hbm, zeros_hbm, out_hbm, idx_v, hist_v):
    c = lax.axis_index("c")
    s = lax.axis_index("s")
    w = s * NC + c
    pltpu.sync_copy(dst_hbm.at[w], idx_v)
    pltpu.sync_copy(zeros_hbm, hist_v)
    ones = jnp.ones((_LANES,), jnp.float32)

    def body(i, carry):
        idx = idx_v[pl.ds(i * _LANES, _LANES)]
        plsc.addupdate_scatter(hist_v, [idx], ones)
        return carry

    lax.fori_loop(0, EPW // _LANES, body, 0)
    pltpu.sync_copy(hist_v, out_hbm.at[w])


def _make_agg(chunk_step):
    n_ch = IDX_BLK // chunk_step

    @functools.partial(
        pl.kernel,
        out_type=jax.ShapeDtypeStruct((NC, NPAD, 128), jnp.float32),
        mesh=_MESH,
        scratch_types=[
            pltpu.VMEM((IDX_BLK, CHUNK), jnp.int32),
            pltpu.VMEM((IDX_BLK, CHUNK), jnp.int32),
            pltpu.VMEM((CHUNK, 128), jnp.float32),
            pltpu.VMEM((CHUNK, 128), jnp.float32),
            pltpu.VMEM_SHARED((NPAD, 128), jnp.float32),
            pltpu.SemaphoreType.DMA,
            pltpu.SemaphoreType.DMA,
        ],
    )
    def agg(src_hbm, dst_hbm, y_hbm, out_hbm,
            src_v, dst_v, rows0, rows1, acc_sh, sem0, sem1):
        c = lax.axis_index("c")
        s = lax.axis_index("s")
        y_c = y_hbm.at[c]
        idx_src = src_hbm.at[s]
        idx_dst = dst_hbm.at[s]
        r0 = s * ROWS_PER_TILE
        pltpu.sync_copy(y_c.at[pl.ds(r0, ROWS_PER_TILE)],
                        acc_sh.at[pl.ds(r0, ROWS_PER_TILE)])
        plsc.subcore_barrier()

        base = c * (chunk_step - 1)

        def ch(k):
            return base + chunk_step * k

        def outer(bi, carry):
            pltpu.sync_copy(idx_src.at[pl.ds(bi * IDX_BLK, IDX_BLK)], src_v)
            pltpu.sync_copy(idx_dst.at[pl.ds(bi * IDX_BLK, IDX_BLK)], dst_v)

            pltpu.async_copy(y_c.at[src_v.at[ch(0)]], rows0, sem0)
            pltpu.async_copy(y_c.at[src_v.at[ch(1)]], rows1, sem1)

            def pair(p, carry2):
                k = 2 * p
                pltpu.make_async_copy(y_c.at[src_v.at[ch(k)]], rows0, sem0).wait()
                pltpu.sync_copy(rows0, acc_sh.at[dst_v.at[ch(k)]], add=True)
                pltpu.async_copy(y_c.at[src_v.at[ch(k + 2)]], rows0, sem0)
                pltpu.make_async_copy(y_c.at[src_v.at[ch(k + 1)]], rows1, sem1).wait()
                pltpu.sync_copy(rows1, acc_sh.at[dst_v.at[ch(k + 1)]], add=True)
                pltpu.async_copy(y_c.at[src_v.at[ch(k + 3)]], rows1, sem1)
                return carry2

            lax.fori_loop(0, n_ch // 2 - 1, pair, 0)
            kl = n_ch - 2
            pltpu.make_async_copy(y_c.at[src_v.at[ch(kl)]], rows0, sem0).wait()
            pltpu.sync_copy(rows0, acc_sh.at[dst_v.at[ch(kl)]], add=True)
            pltpu.make_async_copy(y_c.at[src_v.at[ch(kl + 1)]], rows1, sem1).wait()
            pltpu.sync_copy(rows1, acc_sh.at[dst_v.at[ch(kl + 1)]], add=True)
            return carry

        lax.fori_loop(0, N_IDXBLK, outer, 0)
        plsc.subcore_barrier()
        pltpu.sync_copy(acc_sh.at[pl.ds(r0, ROWS_PER_TILE)],
                        out_hbm.at[c].at[pl.ds(r0, ROWS_PER_TILE)])

    return agg


_agg_l1 = _make_agg(chunk_step=1)
_agg_l2 = _make_agg(chunk_step=2)


_RB = 512
_GRID = NPAD // _RB


def _dis_from(dp_ref):
    deg = jnp.sum(dp_ref[...], axis=0)[:, None] + 1.0
    return lax.rsqrt(deg)


def _xw_body(x_ref, w_ref, dp_ref, o_ref):
    dis = _dis_from(dp_ref)
    xw = jnp.dot(x_ref[...], w_ref[...], preferred_element_type=jnp.float32)
    y = xw * dis
    h = w_ref.shape[1] // 2
    o_ref[0] = y[:, :h]
    o_ref[1] = y[:, h:]


def _mid_body(a_ref, dp_ref, w_ref, b_ref, o_ref):
    dis = _dis_from(dp_ref)
    aggf = jnp.concatenate([a_ref[0], a_ref[1]], axis=1)
    h = jnp.maximum(aggf * dis + b_ref[...], 0.0)
    hw = jnp.dot(h, w_ref[...], preferred_element_type=jnp.float32)
    y2 = hw * dis
    o_ref[0] = y2
    o_ref[1] = y2


def _fin_body(p_ref, y2_ref, dp_ref, b_ref, o_ref):
    dis = _dis_from(dp_ref)
    aggf = p_ref[0] + p_ref[1] - y2_ref[...]
    o_ref[...] = aggf * dis + b_ref[...]


def _tc_xw(x, w, degp, d_out):
    return pl.pallas_call(
        _xw_body,
        grid=(_GRID,),
        in_specs=[
            pl.BlockSpec((_RB, x.shape[1]), lambda i: (i, 0)),
            pl.BlockSpec((w.shape[0], w.shape[1]), lambda i: (0, 0)),
            pl.BlockSpec((NC * NS, _RB), lambda i: (0, i)),
        ],
        out_specs=pl.BlockSpec((2, _RB, d_out // 2), lambda i: (0, i, 0)),
        out_shape=jax.ShapeDtypeStruct((2, NPAD, d_out // 2), jnp.float32),
    )(x, w, degp)


def _tc_mid(agg1, degp, w, b, d_out):
    return pl.pallas_call(
        _mid_body,
        grid=(_GRID,),
        in_specs=[
            pl.BlockSpec((2, _RB, 128), lambda i: (0, i, 0)),
            pl.BlockSpec((NC * NS, _RB), lambda i: (0, i)),
            pl.BlockSpec((w.shape[0], w.shape[1]), lambda i: (0, 0)),
            pl.BlockSpec((1, b.shape[1]), lambda i: (0, 0)),
        ],
        out_specs=pl.BlockSpec((2, _RB, d_out), lambda i: (0, i, 0)),
        out_shape=jax.ShapeDtypeStruct((2, NPAD, d_out), jnp.float32),
    )(agg1, degp, w, b)


def _tc_fin(parts, y2, degp, b):
    d = y2.shape[1]
    return pl.pallas_call(
        _fin_body,
        grid=(_GRID,),
        in_specs=[
            pl.BlockSpec((2, _RB, d), lambda i: (0, i, 0)),
            pl.BlockSpec((_RB, d), lambda i: (i, 0)),
            pl.BlockSpec((NC * NS, _RB), lambda i: (0, i)),
            pl.BlockSpec((1, d), lambda i: (0, 0)),
        ],
        out_specs=pl.BlockSpec((_RB, d), lambda i: (i, 0)),
        out_shape=jax.ShapeDtypeStruct((NPAD, d), jnp.float32),
    )(parts, y2, degp, b)


def kernel(x, edge_index, W1, b1, W2, b2):
    src = edge_index[0].astype(jnp.int32)
    dst = edge_index[1].astype(jnp.int32)
    npad_e = EPAD - N_EDGES
    pad_dst = N_NODES + (jnp.arange(npad_e, dtype=jnp.int32) % (NPAD - N_NODES))
    src_p = jnp.concatenate([src, jnp.zeros((npad_e,), jnp.int32)])
    dst_p = jnp.concatenate([dst, pad_dst])
    src_agg = src_p.reshape(NS, AGG_CHUNKS, CHUNK)
    dst_agg = dst_p.reshape(NS, AGG_CHUNKS, CHUNK)
    dst_deg = dst_p.reshape(NC * NS, EPW)

    zeros1 = jnp.zeros((NPAD,), jnp.float32)
    xp = jnp.zeros((NPAD, IN_CH), x.dtype).at[:N_NODES].set(x)

    degp = _deg_kernel(dst_deg, zeros1)

    y1 = _tc_xw(xp, W1, degp, HID)
    agg1 = _agg_l1(src_agg, dst_agg, y1)
    y2 = _tc_mid(agg1, degp, W2, b1.reshape(1, HID), OUT_CH)
    parts = _agg_l2(src_agg, dst_agg, y2)
    z = _tc_fin(parts, y2[0], degp, b2.reshape(1, OUT_CH))
    return z[:N_NODES]


# speedup vs baseline: 1.2713x; 1.0010x over previous
import functools

import jax
import jax.numpy as jnp
from jax import lax
from jax.experimental import pallas as pl
from jax.experimental.pallas import tpu as pltpu
from jax.experimental.pallas import tpu_sc as plsc

N_NODES = 10000
NPAD = 10240
IN_CH = 128
HID = 256
OUT_CH = 128
N_EDGES = 320000

NC = 2
NS = 16
CHUNK = 128
DEG_CHUNKS = 80
AGG_CHUNKS = 160
IDX_BLK = 40
N_IDXBLK = AGG_CHUNKS // IDX_BLK
EPAD = NC * NS * DEG_CHUNKS * CHUNK
ROWS_PER_TILE = NPAD // NS

_MESH = plsc.VectorSubcoreMesh(core_axis_name="c", subcore_axis_name="s")


EPW = EPAD // (NC * NS)
_LANES = 16


@functools.partial(
    pl.kernel,
    out_type=jax.ShapeDtypeStruct((NC * NS, NPAD), jnp.float32),
    mesh=_MESH,
    scratch_types=[
        pltpu.VMEM((EPW,), jnp.int32),
        pltpu.VMEM((NPAD,), jnp.float32),
    ],
    compiler_params=pltpu.CompilerParams(needs_layout_passes=False),
)
def _deg_kernel(dst_hbm, zeros_hbm, out_hbm, idx_v, hist_v):
    c = lax.axis_index("c")
    s = lax.axis_index("s")
    w = s * NC + c
    pltpu.sync_copy(dst_hbm.at[w], idx_v)
    pltpu.sync_copy(zeros_hbm, hist_v)
    ones = jnp.ones((_LANES,), jnp.float32)

    def body(i, carry):
        idx = idx_v[pl.ds(i * _LANES, _LANES)]
        plsc.addupdate_scatter(hist_v, [idx], ones)
        return carry

    lax.fori_loop(0, EPW // _LANES, body, 0)
    pltpu.sync_copy(hist_v, out_hbm.at[w])


def _make_agg(chunk_step):
    n_ch = IDX_BLK // chunk_step

    @functools.partial(
        pl.kernel,
        out_type=jax.ShapeDtypeStruct((NC, NPAD, 128), jnp.float32),
        mesh=_MESH,
        scratch_types=[
            pltpu.VMEM((IDX_BLK, CHUNK), jnp.int32),
            pltpu.VMEM((IDX_BLK, CHUNK), jnp.int32),
            pltpu.VMEM((CHUNK, 128), jnp.float32),
            pltpu.VMEM((CHUNK, 128), jnp.float32),
            pltpu.VMEM_SHARED((NPAD, 128), jnp.float32),
            pltpu.SemaphoreType.DMA,
            pltpu.SemaphoreType.DMA,
        ],
    )
    def agg(src_hbm, dst_hbm, y_hbm, out_hbm,
            src_v, dst_v, rows0, rows1, acc_sh, sem0, sem1):
        c = lax.axis_index("c")
        s = lax.axis_index("s")
        y_c = y_hbm.at[c]
        idx_src = src_hbm.at[s]
        idx_dst = dst_hbm.at[s]
        r0 = s * ROWS_PER_TILE
        pltpu.sync_copy(y_c.at[pl.ds(r0, ROWS_PER_TILE)],
                        acc_sh.at[pl.ds(r0, ROWS_PER_TILE)])
        plsc.subcore_barrier()

        base = c * (chunk_step - 1)

        def ch(k):
            return base + chunk_step * k

        def outer(bi, carry):
            pltpu.sync_copy(idx_src.at[pl.ds(bi * IDX_BLK, IDX_BLK)], src_v)
            pltpu.sync_copy(idx_dst.at[pl.ds(bi * IDX_BLK, IDX_BLK)], dst_v)

            pltpu.async_copy(y_c.at[src_v.at[ch(0)]], rows0, sem0)
            pltpu.async_copy(y_c.at[src_v.at[ch(1)]], rows1, sem1)

            def pair(p, carry2):
                k = 2 * p
                pltpu.make_async_copy(y_c.at[src_v.at[ch(k)]], rows0, sem0).wait()
                pltpu.sync_copy(rows0, acc_sh.at[dst_v.at[ch(k)]], add=True)
                pltpu.async_copy(y_c.at[src_v.at[ch(k + 2)]], rows0, sem0, priority=1)
                pltpu.make_async_copy(y_c.at[src_v.at[ch(k + 1)]], rows1, sem1).wait()
                pltpu.sync_copy(rows1, acc_sh.at[dst_v.at[ch(k + 1)]], add=True)
                pltpu.async_copy(y_c.at[src_v.at[ch(k + 3)]], rows1, sem1, priority=1)
                return carry2

            lax.fori_loop(0, n_ch // 2 - 1, pair, 0)
            kl = n_ch - 2
            pltpu.make_async_copy(y_c.at[src_v.at[ch(kl)]], rows0, sem0).wait()
            pltpu.sync_copy(rows0, acc_sh.at[dst_v.at[ch(kl)]], add=True)
            pltpu.make_async_copy(y_c.at[src_v.at[ch(kl + 1)]], rows1, sem1).wait()
            pltpu.sync_copy(rows1, acc_sh.at[dst_v.at[ch(kl + 1)]], add=True)
            return carry

        lax.fori_loop(0, N_IDXBLK, outer, 0)
        plsc.subcore_barrier()
        pltpu.sync_copy(acc_sh.at[pl.ds(r0, ROWS_PER_TILE)],
                        out_hbm.at[c].at[pl.ds(r0, ROWS_PER_TILE)])

    return agg


_agg_l1 = _make_agg(chunk_step=1)
_agg_l2 = _make_agg(chunk_step=2)


_RB = 512
_GRID = NPAD // _RB


def _dis_from(dp_ref):
    deg = jnp.sum(dp_ref[...], axis=0)[:, None] + 1.0
    return lax.rsqrt(deg)


def _xw_body(x_ref, w_ref, dp_ref, o_ref):
    dis = _dis_from(dp_ref)
    xw = jnp.dot(x_ref[...], w_ref[...], preferred_element_type=jnp.float32)
    y = xw * dis
    h = w_ref.shape[1] // 2
    o_ref[0] = y[:, :h]
    o_ref[1] = y[:, h:]


def _mid_body(a_ref, dp_ref, w_ref, b_ref, o_ref):
    dis = _dis_from(dp_ref)
    aggf = jnp.concatenate([a_ref[0], a_ref[1]], axis=1)
    h = jnp.maximum(aggf * dis + b_ref[...], 0.0)
    hw = jnp.dot(h, w_ref[...], preferred_element_type=jnp.float32)
    y2 = hw * dis
    o_ref[0] = y2
    o_ref[1] = y2


def _fin_body(p_ref, y2_ref, dp_ref, b_ref, o_ref):
    dis = _dis_from(dp_ref)
    aggf = p_ref[0] + p_ref[1] - y2_ref[...]
    o_ref[...] = aggf * dis + b_ref[...]


def _tc_xw(x, w, degp, d_out):
    return pl.pallas_call(
        _xw_body,
        grid=(_GRID,),
        in_specs=[
            pl.BlockSpec((_RB, x.shape[1]), lambda i: (i, 0)),
            pl.BlockSpec((w.shape[0], w.shape[1]), lambda i: (0, 0)),
            pl.BlockSpec((NC * NS, _RB), lambda i: (0, i)),
        ],
        out_specs=pl.BlockSpec((2, _RB, d_out // 2), lambda i: (0, i, 0)),
        out_shape=jax.ShapeDtypeStruct((2, NPAD, d_out // 2), jnp.float32),
    )(x, w, degp)


def _tc_mid(agg1, degp, w, b, d_out):
    return pl.pallas_call(
        _mid_body,
        grid=(_GRID,),
        in_specs=[
            pl.BlockSpec((2, _RB, 128), lambda i: (0, i, 0)),
            pl.BlockSpec((NC * NS, _RB), lambda i: (0, i)),
            pl.BlockSpec((w.shape[0], w.shape[1]), lambda i: (0, 0)),
            pl.BlockSpec((1, b.shape[1]), lambda i: (0, 0)),
        ],
        out_specs=pl.BlockSpec((2, _RB, d_out), lambda i: (0, i, 0)),
        out_shape=jax.ShapeDtypeStruct((2, NPAD, d_out), jnp.float32),
    )(agg1, degp, w, b)


def _tc_fin(parts, y2, degp, b):
    d = y2.shape[1]
    return pl.pallas_call(
        _fin_body,
        grid=(_GRID,),
        in_specs=[
            pl.BlockSpec((2, _RB, d), lambda i: (0, i, 0)),
            pl.BlockSpec((_RB, d), lambda i: (i, 0)),
            pl.BlockSpec((NC * NS, _RB), lambda i: (0, i)),
            pl.BlockSpec((1, d), lambda i: (0, 0)),
        ],
        out_specs=pl.BlockSpec((_RB, d), lambda i: (i, 0)),
        out_shape=jax.ShapeDtypeStruct((NPAD, d), jnp.float32),
    )(parts, y2, degp, b)


def kernel(x, edge_index, W1, b1, W2, b2):
    src = edge_index[0].astype(jnp.int32)
    dst = edge_index[1].astype(jnp.int32)
    npad_e = EPAD - N_EDGES
    pad_dst = N_NODES + (jnp.arange(npad_e, dtype=jnp.int32) % (NPAD - N_NODES))
    src_p = jnp.concatenate([src, jnp.zeros((npad_e,), jnp.int32)])
    dst_p = jnp.concatenate([dst, pad_dst])
    src_agg = src_p.reshape(NS, AGG_CHUNKS, CHUNK)
    dst_agg = dst_p.reshape(NS, AGG_CHUNKS, CHUNK)
    dst_deg = dst_p.reshape(NC * NS, EPW)

    zeros1 = jnp.zeros((NPAD,), jnp.float32)
    xp = jnp.zeros((NPAD, IN_CH), x.dtype).at[:N_NODES].set(x)

    degp = _deg_kernel(dst_deg, zeros1)

    y1 = _tc_xw(xp, W1, degp, HID)
    agg1 = _agg_l1(src_agg, dst_agg, y1)
    y2 = _tc_mid(agg1, degp, W2, b1.reshape(1, HID), OUT_CH)
    parts = _agg_l2(src_agg, dst_agg, y2)
    z = _tc_fin(parts, y2[0], degp, b2.reshape(1, OUT_CH))
    return z[:N_NODES]
